# Initial kernel scaffold; baseline (speedup 1.0000x reference)
import functools

import jax
import jax.numpy as jnp
from jax import lax
from jax.experimental import pallas as pl
from jax.experimental.pallas import tpu as pltpu
from jax.experimental.pallas import tpu_sc as plsc

N = 10000
E = 320000
D = 128
L = 5
A_TABLES = 9
A_ROWS = 124
B_ROWS = 32

NC = 2
NS = 16
NW = NC * NS

EC = E // 128
ATOM_CHUNKS = 79
NP = ATOM_CHUNKS * 128
ROWS_PER_TILE = N // NS

_mesh = plsc.VectorSubcoreMesh(core_axis_name="c", subcore_axis_name="s")


@functools.partial(
    pl.kernel,
    mesh=_mesh,
    out_type=jax.ShapeDtypeStruct((NP, D), jnp.float32),
    scratch_types=[
        pltpu.VMEM((128,), jnp.int32),
        pltpu.VMEM((128, D), jnp.float32),
        pltpu.VMEM((128, D), jnp.float32),
    ],
)
def _atom_encode_sc(xoff_hbm, atab_hbm, h_hbm, idx_v, tbuf, hacc):
    wid = lax.axis_index("s") * NC + lax.axis_index("c")

    def chunk_body(j, carry):
        cid = wid + NW * j

        @pl.when(cid < ATOM_CHUNKS)
        def _():
            base = cid * 128
            pltpu.sync_copy(xoff_hbm.at[0, pl.ds(base, 128)], idx_v)
            pltpu.sync_copy(atab_hbm.at[idx_v], hacc)
            for i in range(1, A_TABLES):
                pltpu.sync_copy(xoff_hbm.at[i, pl.ds(base, 128)], idx_v)
                pltpu.sync_copy(atab_hbm.at[idx_v], tbuf)

                def row_body(r, c2):
                    for cc in range(D // 16):
                        sl = pl.ds(cc * 16, 16)
                        hacc[r, sl] = hacc[r, sl] + tbuf[r, sl]
                    return c2

                lax.fori_loop(0, 128, row_body, 0)
            pltpu.sync_copy(hacc, h_hbm.at[pl.ds(base, 128)])

        return carry

    lax.fori_loop(0, (ATOM_CHUNKS + NW - 1) // NW, chunk_body, 0)


@functools.partial(
    pl.kernel,
    mesh=_mesh,
    out_type=jax.ShapeDtypeStruct((NC, N, D), jnp.float32),
    scratch_types=[
        pltpu.VMEM((128,), jnp.int32),
        pltpu.VMEM((128,), jnp.int32),
        pltpu.VMEM((128,), jnp.int32),
        pltpu.VMEM((128, D), jnp.float32),
        pltpu.VMEM((128, D), jnp.float32),
        pltpu.VMEM((125, D), jnp.float32),
        pltpu.VMEM_SHARED((N, D), jnp.float32),
    ],
)
def _edge_aggregate_sc(h_hbm, ctab_hbm, src_hbm, dst_hbm, cidx_hbm, out_hbm,
                       src_v, dst_v, cidx_v, hbuf, ebuf, zbuf, aggr_sh):
    c_i = lax.axis_index("c")
    s_i = lax.axis_index("s")
    wid = s_i * NC + c_i

    def zrow(r, carry):
        for cc in range(D // 16):
            zbuf[r, pl.ds(cc * 16, 16)] = jnp.zeros((16,), jnp.float32)
        return carry

    lax.fori_loop(0, 125, zrow, 0)
    for k in range(ROWS_PER_TILE // 125):
        pltpu.sync_copy(zbuf, aggr_sh.at[pl.ds(s_i * ROWS_PER_TILE + k * 125, 125)])
    plsc.subcore_barrier()

    def chunk_body(j, carry):
        cid = wid + NW * j

        @pl.when(cid < EC)
        def _():
            pltpu.sync_copy(src_hbm.at[cid], src_v)
            pltpu.sync_copy(dst_hbm.at[cid], dst_v)
            pltpu.sync_copy(cidx_hbm.at[cid], cidx_v)
            pltpu.sync_copy(h_hbm.at[src_v], hbuf)
            pltpu.sync_copy(ctab_hbm.at[cidx_v], ebuf)

            def row_body(r, c2):
                for cc in range(D // 16):
                    sl = pl.ds(cc * 16, 16)
                    ebuf[r, sl] = jnp.maximum(hbuf[r, sl] + ebuf[r, sl], 0.0)
                return c2

            lax.fori_loop(0, 128, row_body, 0)
            pltpu.sync_copy(ebuf, aggr_sh.at[dst_v], add=True)

        return carry

    lax.fori_loop(0, (EC + NW - 1) // NW, chunk_body, 0)
    plsc.subcore_barrier()
    pltpu.sync_copy(aggr_sh.at[pl.ds(s_i * ROWS_PER_TILE, ROWS_PER_TILE)],
                    out_hbm.at[c_i, pl.ds(s_i * ROWS_PER_TILE, ROWS_PER_TILE)])


def _ctab_body(b0_ref, b12_ref, out_ref):
    b1 = b12_ref[0, 0]
    b2 = b12_ref[0, 1]
    m = b1[:, None, :] + b2[None, :, :]
    out_ref[0] = m.reshape(B_ROWS * B_ROWS, D) + b0_ref[0, 0]


def _build_ctab(bond_tables):
    return pl.pallas_call(
        _ctab_body,
        grid=(L, B_ROWS),
        in_specs=[
            pl.BlockSpec((1, 1, 1, D), lambda l, i: (l, 0, i, 0)),
            pl.BlockSpec((1, 2, B_ROWS, D), lambda l, i: (l, 1, 0, 0)),
        ],
        out_specs=pl.BlockSpec((1, B_ROWS * B_ROWS, D), lambda l, i: (l, i, 0)),
        out_shape=jax.ShapeDtypeStruct((L, B_ROWS * B_ROWS * B_ROWS, D),
                                       jnp.float32),
    )(bond_tables, bond_tables)


_NB = 10
_BN = N // _NB


def _mlp1_body(coef_ref, h_ref, p_ref, w_ref, b_ref, y_ref, st_ref):
    z = coef_ref[0, 0] * h_ref[...] + p_ref[0] + p_ref[1]
    y = jnp.dot(z, w_ref[...], preferred_element_type=jnp.float32) + b_ref[...]
    y_ref[...] = y

    @pl.when(pl.program_id(0) == 0)
    def _():
        st_ref[...] = jnp.zeros((2, D), jnp.float32)

    s1 = jnp.sum(y, axis=0)
    s2 = jnp.sum(y * y, axis=0)
    st_ref[...] = st_ref[...] + jnp.stack([s1, s2])


def _mlp_pass1(h, parts, w1, b1v, coef):
    return pl.pallas_call(
        _mlp1_body,
        grid=(_NB,),
        in_specs=[
            pl.BlockSpec(memory_space=pltpu.SMEM),
            pl.BlockSpec((_BN, D), lambda i: (i, 0)),
            pl.BlockSpec((NC, _BN, D), lambda i: (0, i, 0)),
            pl.BlockSpec((D, D), lambda i: (0, 0)),
            pl.BlockSpec((1, D), lambda i: (0, 0)),
        ],
        out_specs=[
            pl.BlockSpec((_BN, D), lambda i: (i, 0)),
            pl.BlockSpec((2, D), lambda i: (0, 0)),
        ],
        out_shape=[
            jax.ShapeDtypeStruct((N, D), jnp.float32),
            jax.ShapeDtypeStruct((2, D), jnp.float32),
        ],
    )(coef, h, parts, w1, b1v)


def _mlp2_body(y_ref, st_ref, g_ref, bt_ref, w_ref, b_ref, o_ref, *, last):
    mu = st_ref[0:1, :] * (1.0 / N)
    ex2 = st_ref[1:2, :] * (1.0 / N)
    var = ex2 - mu * mu
    inv = lax.rsqrt(var + 1e-5) * g_ref[...]
    z = (y_ref[...] - mu) * inv + bt_ref[...]
    z = jnp.maximum(z, 0.0)
    o = jnp.dot(z, w_ref[...], preferred_element_type=jnp.float32) + b_ref[...]
    if not last:
        o = jnp.maximum(o, 0.0)
    o_ref[...] = o


def _mlp_pass2(y, stats, gv, btv, w2, b2v, last):
    return pl.pallas_call(
        functools.partial(_mlp2_body, last=last),
        grid=(_NB,),
        in_specs=[
            pl.BlockSpec((_BN, D), lambda i: (i, 0)),
            pl.BlockSpec((2, D), lambda i: (0, 0)),
            pl.BlockSpec((1, D), lambda i: (0, 0)),
            pl.BlockSpec((1, D), lambda i: (0, 0)),
            pl.BlockSpec((D, D), lambda i: (0, 0)),
            pl.BlockSpec((1, D), lambda i: (0, 0)),
        ],
        out_specs=pl.BlockSpec((_BN, D), lambda i: (i, 0)),
        out_shape=jax.ShapeDtypeStruct((N, D), jnp.float32),
    )(y, stats, gv, btv, w2, b2v)


def kernel(x, edge_index, edge_attr, atom_tables, bond_tables,
           W1, b1, gamma, beta, W2, b2, eps):
    xoff = x.astype(jnp.int32) + jnp.arange(A_TABLES, dtype=jnp.int32)[None, :] * A_ROWS
    xoff_t = jnp.zeros((A_TABLES, NP), jnp.int32).at[:, :N].set(xoff.T)
    atab = atom_tables.reshape(A_TABLES * A_ROWS, D)

    src_c = edge_index[0].astype(jnp.int32).reshape(EC, 128)
    dst_c = edge_index[1].astype(jnp.int32).reshape(EC, 128)
    ea = edge_attr.astype(jnp.int32)
    cidx_c = (ea[:, 0] * (B_ROWS * B_ROWS) + ea[:, 1] * B_ROWS
              + ea[:, 2]).reshape(EC, 128)

    ctab = _build_ctab(bond_tables)

    h = _atom_encode_sc(xoff_t, atab)[:N]

    for l in range(L):
        parts = _edge_aggregate_sc(h, ctab[l], src_c, dst_c, cidx_c)
        coef = (1.0 + eps[l]).reshape(1, 1)
        y, stats = _mlp_pass1(h, parts, W1[l], b1[l].reshape(1, D), coef)
        h = _mlp_pass2(y, stats, gamma[l].reshape(1, D), beta[l].reshape(1, D),
                       W2[l], b2[l].reshape(1, D), last=(l == L - 1))
    return h


# speedup vs baseline: 5.5645x; 5.5645x over previous
import functools

import jax
import jax.numpy as jnp
from jax import lax
from jax.experimental import pallas as pl
from jax.experimental.pallas import tpu as pltpu
from jax.experimental.pallas import tpu_sc as plsc

N = 10000
E = 320000
D = 128
L = 5
A_TABLES = 9
A_ROWS = 124
B_ROWS = 32

NC = 2
NS = 16
NW = NC * NS

EC = E // 128
ATOM_CHUNKS = 79
NP = ATOM_CHUNKS * 128
ROWS_PER_TILE = N // NS

_mesh = plsc.VectorSubcoreMesh(core_axis_name="c", subcore_axis_name="s")


@functools.partial(
    pl.kernel,
    mesh=_mesh,
    out_type=jax.ShapeDtypeStruct((NP, D), jnp.float32),
    scratch_types=[
        pltpu.VMEM((128,), jnp.int32),
        pltpu.VMEM((128, D), jnp.float32),
        pltpu.VMEM((128, D), jnp.float32),
    ],
)
def _atom_encode_sc(xoff_hbm, atab_hbm, h_hbm, idx_v, tbuf, hacc):
    wid = lax.axis_index("s") * NC + lax.axis_index("c")

    def chunk_body(j, carry):
        cid = wid + NW * j

        @pl.when(cid < ATOM_CHUNKS)
        def _():
            base = cid * 128
            pltpu.sync_copy(xoff_hbm.at[pl.ds(base, 128)], idx_v)
            pltpu.sync_copy(atab_hbm.at[idx_v], hacc)
            for i in range(1, A_TABLES):
                pltpu.sync_copy(xoff_hbm.at[pl.ds(i * NP + base, 128)], idx_v)
                pltpu.sync_copy(atab_hbm.at[idx_v], tbuf)

                def row_body(r, c2):
                    for cc in range(D // 16):
                        sl = pl.ds(cc * 16, 16)
                        hacc[r, sl] = hacc[r, sl] + tbuf[r, sl]
                    return c2

                lax.fori_loop(0, 128, row_body, 0)
            pltpu.sync_copy(hacc, h_hbm.at[pl.ds(base, 128)])

        return carry

    lax.fori_loop(0, (ATOM_CHUNKS + NW - 1) // NW, chunk_body, 0)


@functools.partial(
    pl.kernel,
    mesh=_mesh,
    out_type=jax.ShapeDtypeStruct((NC, N, D), jnp.float32),
    scratch_types=[
        pltpu.VMEM((128,), jnp.int32),
        pltpu.VMEM((128,), jnp.int32),
        pltpu.VMEM((128,), jnp.int32),
        pltpu.VMEM((128, D), jnp.float32),
        pltpu.VMEM((128, D), jnp.float32),
        pltpu.VMEM((128, D), jnp.float32),
        pltpu.VMEM_SHARED((N, D), jnp.float32),
    ],
)
def _edge_aggregate_sc(h_hbm, ctab_hbm, src_hbm, dst_hbm, cidx_hbm, out_hbm,
                       src_v, dst_v, cidx_v, hbuf, ebuf, zbuf, aggr_sh):
    c_i = lax.axis_index("c")
    s_i = lax.axis_index("s")
    wid = s_i * NC + c_i

    def zrow(r, carry):
        for cc in range(D // 16):
            zbuf[r, pl.ds(cc * 16, 16)] = jnp.zeros((16,), jnp.float32)
        return carry

    lax.fori_loop(0, 128, zrow, 0)

    def zchunk(j, carry):
        zid = s_i + NS * j

        @pl.when(zid < N // 128)
        def _():
            pltpu.sync_copy(zbuf, aggr_sh.at[pl.ds(zid * 128, 128)])

        return carry

    lax.fori_loop(0, (N // 128 + NS - 1) // NS, zchunk, 0)

    @pl.when(s_i == 0)
    def _():
        pltpu.sync_copy(zbuf.at[pl.ds(0, N % 128)],
                        aggr_sh.at[pl.ds((N // 128) * 128, N % 128)])

    plsc.subcore_barrier()

    def chunk_body(j, carry):
        cid = wid + NW * j

        @pl.when(cid < EC)
        def _():
            base = cid * 128
            pltpu.sync_copy(src_hbm.at[pl.ds(base, 128)], src_v)
            pltpu.sync_copy(dst_hbm.at[pl.ds(base, 128)], dst_v)
            pltpu.sync_copy(cidx_hbm.at[pl.ds(base, 128)], cidx_v)
            pltpu.sync_copy(h_hbm.at[src_v], hbuf)
            pltpu.sync_copy(ctab_hbm.at[cidx_v], ebuf)

            def row_body(r, c2):
                for cc in range(D // 16):
                    sl = pl.ds(cc * 16, 16)
                    ebuf[r, sl] = jnp.maximum(hbuf[r, sl] + ebuf[r, sl], 0.0)
                return c2

            lax.fori_loop(0, 128, row_body, 0)
            pltpu.sync_copy(ebuf, aggr_sh.at[dst_v], add=True)

        return carry

    lax.fori_loop(0, (EC + NW - 1) // NW, chunk_body, 0)
    plsc.subcore_barrier()

    def wchunk(j, carry):
        zid = s_i + NS * j

        @pl.when(zid < N // 128)
        def _():
            pltpu.sync_copy(aggr_sh.at[pl.ds(zid * 128, 128)],
                            out_hbm.at[c_i, pl.ds(zid * 128, 128)])

        return carry

    lax.fori_loop(0, (N // 128 + NS - 1) // NS, wchunk, 0)

    @pl.when(s_i == 0)
    def _():
        pltpu.sync_copy(aggr_sh.at[pl.ds((N // 128) * 128, N % 128)],
                        out_hbm.at[c_i, pl.ds((N // 128) * 128, N % 128)])


def _ctab_body(bt_ref, out_ref):
    b1 = bt_ref[0, 1]
    b2 = bt_ref[0, 2]
    i = pl.program_id(1)
    b0row = bt_ref[0, 0, pl.ds(i, 1)]
    m = b1[:, None, :] + b2[None, :, :]
    out_ref[0] = m.reshape(B_ROWS * B_ROWS, D) + b0row


def _build_ctab(bond_tables):
    return pl.pallas_call(
        _ctab_body,
        grid=(L, B_ROWS),
        in_specs=[
            pl.BlockSpec((1, 3, B_ROWS, D), lambda l, i: (l, 0, 0, 0)),
        ],
        out_specs=pl.BlockSpec((1, B_ROWS * B_ROWS, D), lambda l, i: (l, i, 0)),
        out_shape=jax.ShapeDtypeStruct((L, B_ROWS * B_ROWS * B_ROWS, D),
                                       jnp.float32),
    )(bond_tables)


_NB = 10
_BN = N // _NB


def _mlp1_body(coef_ref, h_ref, p_ref, w_ref, b_ref, y_ref, st_ref):
    z = coef_ref[0, 0] * h_ref[...] + p_ref[0] + p_ref[1]
    y = jnp.dot(z, w_ref[...], preferred_element_type=jnp.float32) + b_ref[...]
    y_ref[...] = y

    @pl.when(pl.program_id(0) == 0)
    def _():
        st_ref[...] = jnp.zeros((1, D), jnp.float32)

    st_ref[...] = st_ref[...] + jnp.sum(y, axis=0)[None]


def _mlp_pass1(h, parts, w1, b1v, coef):
    return pl.pallas_call(
        _mlp1_body,
        grid=(_NB,),
        in_specs=[
            pl.BlockSpec(memory_space=pltpu.SMEM),
            pl.BlockSpec((_BN, D), lambda i: (i, 0)),
            pl.BlockSpec((NC, _BN, D), lambda i: (0, i, 0)),
            pl.BlockSpec((D, D), lambda i: (0, 0)),
            pl.BlockSpec((1, D), lambda i: (0, 0)),
        ],
        out_specs=[
            pl.BlockSpec((_BN, D), lambda i: (i, 0)),
            pl.BlockSpec((1, D), lambda i: (0, 0)),
        ],
        out_shape=[
            jax.ShapeDtypeStruct((N, D), jnp.float32),
            jax.ShapeDtypeStruct((1, D), jnp.float32),
        ],
    )(coef, h, parts, w1, b1v)


def _var_body(y_ref, s_ref, v_ref):
    d = y_ref[...] - s_ref[...] * (1.0 / N)

    @pl.when(pl.program_id(0) == 0)
    def _():
        v_ref[...] = jnp.zeros((1, D), jnp.float32)

    v_ref[...] = v_ref[...] + jnp.sum(d * d, axis=0)[None]


def _var_pass(y, sums):
    return pl.pallas_call(
        _var_body,
        grid=(_NB,),
        in_specs=[
            pl.BlockSpec((_BN, D), lambda i: (i, 0)),
            pl.BlockSpec((1, D), lambda i: (0, 0)),
        ],
        out_specs=pl.BlockSpec((1, D), lambda i: (0, 0)),
        out_shape=jax.ShapeDtypeStruct((1, D), jnp.float32),
    )(y, sums)


def _mlp2_body(y_ref, s_ref, v_ref, g_ref, bt_ref, w_ref, b_ref, o_ref, *, last):
    mu = s_ref[...] * (1.0 / N)
    var = v_ref[...] * (1.0 / N)
    inv = lax.rsqrt(var + 1e-5) * g_ref[...]
    z = (y_ref[...] - mu) * inv + bt_ref[...]
    z = jnp.maximum(z, 0.0)
    o = jnp.dot(z, w_ref[...], preferred_element_type=jnp.float32) + b_ref[...]
    if not last:
        o = jnp.maximum(o, 0.0)
    o_ref[...] = o


def _mlp_pass2(y, sums, ssq, gv, btv, w2, b2v, last):
    return pl.pallas_call(
        functools.partial(_mlp2_body, last=last),
        grid=(_NB,),
        in_specs=[
            pl.BlockSpec((_BN, D), lambda i: (i, 0)),
            pl.BlockSpec((1, D), lambda i: (0, 0)),
            pl.BlockSpec((1, D), lambda i: (0, 0)),
            pl.BlockSpec((1, D), lambda i: (0, 0)),
            pl.BlockSpec((1, D), lambda i: (0, 0)),
            pl.BlockSpec((D, D), lambda i: (0, 0)),
            pl.BlockSpec((1, D), lambda i: (0, 0)),
        ],
        out_specs=pl.BlockSpec((_BN, D), lambda i: (i, 0)),
        out_shape=jax.ShapeDtypeStruct((N, D), jnp.float32),
    )(y, sums, ssq, gv, btv, w2, b2v)


def kernel(x, edge_index, edge_attr, atom_tables, bond_tables,
           W1, b1, gamma, beta, W2, b2, eps):
    xoff = x.astype(jnp.int32) + jnp.arange(A_TABLES, dtype=jnp.int32)[None, :] * A_ROWS
    xoff_t = jnp.zeros((A_TABLES, NP), jnp.int32).at[:, :N].set(xoff.T).reshape(-1)
    atab = atom_tables.reshape(A_TABLES * A_ROWS, D)

    src_c = edge_index[0].astype(jnp.int32)
    dst_c = edge_index[1].astype(jnp.int32)
    ea = edge_attr.astype(jnp.int32)
    cidx_c = ea[:, 0] * (B_ROWS * B_ROWS) + ea[:, 1] * B_ROWS + ea[:, 2]

    ctab = _build_ctab(bond_tables)

    h = _atom_encode_sc(xoff_t, atab)[:N]

    for l in range(L):
        parts = _edge_aggregate_sc(h, ctab[l], src_c, dst_c, cidx_c)
        coef = (1.0 + eps[l]).reshape(1, 1)
        y, sums = _mlp_pass1(h, parts, W1[l], b1[l].reshape(1, D), coef)
        ssq = _var_pass(y, sums)
        h = _mlp_pass2(y, sums, ssq, gamma[l].reshape(1, D),
                       beta[l].reshape(1, D), W2[l], b2[l].reshape(1, D),
                       last=(l == L - 1))
    return h
